# baseline (speedup 1.0000x reference)
import functools

import jax
import jax.numpy as jnp
from jax.experimental import pallas as pl

_K = 32
_LANES = 128
_TOPJ = 4
_BISECT_ITERS = 21
_BLOCK_ROWS = 200
_SUB_ROWS = 40


def _lane_topj(a_ref, r0, t, nf, rem, n):
    L = _LANES
    neg = jnp.asarray(-jnp.inf, a_ref.dtype)
    nseed = min(_TOPJ, nf)
    S = [a_ref[r0:r0 + t, c * L:(c + 1) * L] for c in range(nseed)]
    S += [jnp.full((t, L), neg, a_ref.dtype) for _ in range(_TOPJ - nseed)]

    def insert(v):
        for j in range(_TOPJ):
            top = jnp.maximum(S[j], v)
            if j < _TOPJ - 1:
                v = jnp.minimum(S[j], v)
            S[j] = top

    for c in range(nseed, nf):
        insert(a_ref[r0:r0 + t, c * L:(c + 1) * L])
    if rem:
        tail = a_ref[r0:r0 + t, nf * L:n]
        pad = jnp.full((t, L - rem), neg, a_ref.dtype)
        insert(jnp.concatenate([tail, pad], axis=1))
    return jnp.concatenate(S, axis=1)


def _topk_mask_body(a_ref, o_ref, *, k, iters):
    rblk, n = a_ref.shape
    L = _LANES
    nf = n // L
    rem = n - nf * L
    t = _SUB_ROWS if rblk % _SUB_ROWS == 0 else rblk

    cand = jnp.concatenate(
        [_lane_topj(a_ref, r0, t, nf, rem, n) for r0 in range(0, rblk, t)],
        axis=0)

    cmax = jnp.max(cand, axis=1, keepdims=True)
    hi = jnp.maximum(cmax, 0.0) * (1.0 + 1e-4) + 1e-20
    lo = jnp.zeros_like(hi)

    def step(_, bracket):
        lo, hi = bracket
        m = 0.5 * (lo + hi)
        c = jnp.sum(jnp.where(cand >= m, 1.0, 0.0), axis=1, keepdims=True)
        ge = c >= k
        return jnp.where(ge, m, lo), jnp.where(ge, hi, m)

    lo, hi = jax.lax.fori_loop(0, iters, step, (lo, hi))
    av = a_ref[...]
    o_ref[...] = jnp.where(av >= lo, av, 0.0)


def kernel(idx, A):
    del idx
    n, m = A.shape
    block_rows = _BLOCK_ROWS if n % _BLOCK_ROWS == 0 else n
    grid = (n // block_rows,)
    body = functools.partial(_topk_mask_body, k=_K, iters=_BISECT_ITERS)
    return pl.pallas_call(
        body,
        grid=grid,
        in_specs=[pl.BlockSpec((block_rows, m), lambda i: (i, 0))],
        out_specs=pl.BlockSpec((block_rows, m), lambda i: (i, 0)),
        out_shape=jax.ShapeDtypeStruct((n, m), A.dtype),
    )(A)


# speedup vs baseline: 1.0493x; 1.0493x over previous
import functools

import jax
import jax.numpy as jnp
from jax.experimental import pallas as pl

_K = 32
_LANES = 128
_TOPJ = 4
_BISECT_ITERS = 18
_BLOCK_ROWS = 200
_SUB_ROWS = 40


def _cmp_exchange(S, i, j):
    hi_ = jnp.maximum(S[i], S[j])
    lo_ = jnp.minimum(S[i], S[j])
    S[i], S[j] = hi_, lo_


def _lane_topj(a_ref, r0, t, nf, rem, n):
    L = _LANES
    neg = jnp.asarray(-jnp.inf, a_ref.dtype)

    def chunk(c):
        return a_ref[r0:r0 + t, c * L:(c + 1) * L]

    S = [chunk(c) for c in range(_TOPJ)]
    _cmp_exchange(S, 0, 1)
    _cmp_exchange(S, 2, 3)
    _cmp_exchange(S, 0, 2)
    _cmp_exchange(S, 1, 3)
    _cmp_exchange(S, 1, 2)

    def insert_pair(va, vb):
        t0 = jnp.maximum(va, vb)
        t1 = jnp.minimum(va, vb)
        S[2] = jnp.maximum(S[2], t1)
        S[3] = jnp.maximum(S[3], t0)
        _cmp_exchange(S, 0, 2)
        _cmp_exchange(S, 1, 3)
        _cmp_exchange(S, 0, 1)
        _cmp_exchange(S, 2, 3)

    rest = list(range(_TOPJ, nf))
    if rem:
        tail = a_ref[r0:r0 + t, nf * L:n]
        pad = jnp.full((t, L - rem), neg, a_ref.dtype)
        tail_chunk = jnp.concatenate([tail, pad], axis=1)
    else:
        tail_chunk = None

    while len(rest) >= 2:
        insert_pair(chunk(rest.pop(0)), chunk(rest.pop(0)))
    leftover = chunk(rest.pop(0)) if rest else None
    if tail_chunk is not None and leftover is not None:
        insert_pair(leftover, tail_chunk)
    elif tail_chunk is not None or leftover is not None:
        v = tail_chunk if tail_chunk is not None else leftover
        for j in range(_TOPJ):
            top = jnp.maximum(S[j], v)
            if j < _TOPJ - 1:
                v = jnp.minimum(S[j], v)
            S[j] = top
    return jnp.concatenate(S, axis=1)


def _topk_mask_body(a_ref, o_ref, *, k, iters):
    rblk, n = a_ref.shape
    L = _LANES
    nf = n // L
    rem = n - nf * L
    t = _SUB_ROWS if rblk % _SUB_ROWS == 0 else rblk

    cand = jnp.concatenate(
        [_lane_topj(a_ref, r0, t, nf, rem, n) for r0 in range(0, rblk, t)],
        axis=0)

    cmax = jnp.max(cand, axis=1, keepdims=True)
    hi = jnp.maximum(cmax, 0.0) * (1.0 + 1e-4) + 1e-20
    lo = jnp.zeros_like(hi)

    def step(_, bracket):
        lo, hi = bracket
        m = 0.5 * (lo + hi)
        c = jnp.sum(jnp.where(cand >= m, 1.0, 0.0), axis=1, keepdims=True)
        ge = c >= k
        return jnp.where(ge, m, lo), jnp.where(ge, hi, m)

    lo, hi = jax.lax.fori_loop(0, iters, step, (lo, hi))
    av = a_ref[...]
    o_ref[...] = jnp.where(av >= lo, av, 0.0)


def kernel(idx, A):
    del idx
    n, m = A.shape
    block_rows = _BLOCK_ROWS if n % _BLOCK_ROWS == 0 else n
    grid = (n // block_rows,)
    body = functools.partial(_topk_mask_body, k=_K, iters=_BISECT_ITERS)
    return pl.pallas_call(
        body,
        grid=grid,
        in_specs=[pl.BlockSpec((block_rows, m), lambda i: (i, 0))],
        out_specs=pl.BlockSpec((block_rows, m), lambda i: (i, 0)),
        out_shape=jax.ShapeDtypeStruct((n, m), A.dtype),
    )(A)
